# baseline (speedup 1.0000x reference)
import functools

import jax
import jax.numpy as jnp
from jax import lax
from jax.experimental import pallas as pl
from jax.experimental.pallas import tpu as pltpu
from jax.experimental.pallas import tpu_sc as plsc

F32 = jnp.float32
NC, NS = 2, 16
EW = 80

def _dot(a, b):
    return jax.lax.dot_general(a, b, (((1,), (0,)), ((), ())),
                               preferred_element_type=F32)


def _dot_t(a, bT):
    return jax.lax.dot_general(a, bT, (((1,), (1,)), ((), ())),
                               preferred_element_type=F32)


def _sc_degree(packed3, zeros_n, n):
    nwt = packed3.shape[1]

    mesh = plsc.VectorSubcoreMesh(core_axis_name="c", subcore_axis_name="s",
                                  num_cores=NC, num_subcores=NS)

    @functools.partial(
        pl.kernel, mesh=mesh,
        out_type=jax.ShapeDtypeStruct((NC, 1, n), F32),
        scratch_types=[
            pltpu.VMEM((nwt, EW), jnp.int32),
            pltpu.VMEM((EW,), F32),
            pltpu.VMEM((EW,), jnp.int32),
            pltpu.VMEM((EW,), jnp.int32),
            pltpu.VMEM_SHARED((n,), F32),
            pltpu.SemaphoreType.DMA,
            pltpu.SemaphoreType.DMA,
        ],
    )
    def k(pk_hbm, zeros_hbm, out_hbm, pk, ones_v, db0, db1, acc, sem0, sem1):
        cid = lax.axis_index("c")
        sid = lax.axis_index("s")

        for kk in range(EW // 16):
            ones_v[pl.ds(kk * 16, 16)] = jnp.ones((16,), F32)
        wid = cid * NS + sid
        pltpu.sync_copy(pk_hbm.at[wid], pk)

        @pl.when(sid == 0)
        def _():
            pltpu.sync_copy(zeros_hbm, acc)
        plsc.subcore_barrier()

        dbs = (db0, db1)
        sems = (sem0, sem1)

        def s_launch(b, j):
            for kk in range(EW // 16):
                dbs[b][pl.ds(kk * 16, 16)] = lax.shift_right_logical(
                    pk[j, pl.ds(kk * 16, 16)], 16)
            pltpu.async_copy(ones_v, acc.at[dbs[b]], sems[b], add=True)

        def s_wait(b):
            pltpu.make_async_copy(ones_v, acc.at[dbs[b]], sems[b]).wait()

        s_launch(0, 0)
        s_launch(1, 1)

        def body(t, carry):
            j = 2 * t
            for b in range(2):
                s_wait(b)

                @pl.when(j + b + 2 < nwt)
                def _():
                    s_launch(b, j + b + 2)
            return carry

        lax.fori_loop(0, nwt // 2, body, 0)
        if nwt % 2 == 1:
            s_wait(0)
        plsc.subcore_barrier()

        @pl.when(sid == 0)
        def _():
            pltpu.sync_copy(acc, out_hbm.at[cid, 0])

    return k(packed3, zeros_n)


def _sc_aggregate(packed3, v, n, h):
    nwt = packed3.shape[1]
    wb = (n // NS) & ~7
    wb_last = n - (NS - 1) * wb
    NB = 3

    mesh = plsc.VectorSubcoreMesh(core_axis_name="c", subcore_axis_name="s",
                                  num_cores=NC, num_subcores=NS)

    @functools.partial(
        pl.kernel, mesh=mesh,
        out_type=jax.ShapeDtypeStruct((NC, n, h), F32),
        scratch_types=[
            pltpu.VMEM((nwt, EW), jnp.int32),
            *[pltpu.VMEM((EW,), jnp.int32) for _ in range(2 * NB)],
            *[pltpu.VMEM((EW, h), F32) for _ in range(NB)],
            pltpu.VMEM_SHARED((n, h), F32),
            *[pltpu.SemaphoreType.DMA for _ in range(2 * NB)],
        ],
    )
    def k(pk_hbm, v_hbm, out_hbm, pk, *bufs):
        sbs = bufs[0:2 * NB:2]
        dbs = bufs[1:2 * NB:2]
        rows = bufs[2 * NB:3 * NB]
        acc = bufs[3 * NB]
        sems = bufs[3 * NB + 1:3 * NB + 1 + NB]
        ssems = bufs[3 * NB + 1 + NB:]
        cid = lax.axis_index("c")
        sid = lax.axis_index("s")
        wid = cid * NS + sid
        pltpu.sync_copy(pk_hbm.at[wid], pk)

        def zbody(r, c):
            for kk in range(h // 16):
                rows[0][r, pl.ds(kk * 16, 16)] = jnp.zeros((16,), F32)
            return c

        lax.fori_loop(0, EW, zbody, 0)
        zc = 48
        base = sid * wb
        for c in range(wb // zc):
            pltpu.sync_copy(rows[0].at[pl.ds(0, zc)],
                            acc.at[pl.ds(base + c * zc, zc)])
        if wb % zc:
            pltpu.sync_copy(rows[0].at[pl.ds(0, wb % zc)],
                            acc.at[pl.ds(base + (wb // zc) * zc, wb % zc)])

        @pl.when(sid == NS - 1)
        def _():
            ext = wb_last - wb
            pltpu.sync_copy(rows[0].at[pl.ds(0, ext)],
                            acc.at[pl.ds(NS * wb, ext)])
        plsc.subcore_barrier()

        def unpack(j, sb, db):
            for kk in range(EW // 16):
                pv = pk[j, pl.ds(kk * 16, 16)]
                sb[pl.ds(kk * 16, 16)] = pv & 0xFFFF
                db[pl.ds(kk * 16, 16)] = lax.shift_right_logical(pv, 16)

        def g_start(b, j):
            unpack(j, sbs[b], dbs[b])
            pltpu.make_async_copy(v_hbm.at[sbs[b]], rows[b], sems[b]).start()

        def g_wait(b):
            pltpu.make_async_copy(v_hbm.at[sbs[b]], rows[b], sems[b]).wait()

        for b in range(NB):
            g_start(b, b)

        def body(t, carry):
            j = NB * t
            descs = []
            for b in range(NB):
                g_wait(b)
                descs.append(pltpu.async_copy(rows[b], acc.at[dbs[b]],
                                              ssems[b], add=True))
            for b in range(NB):
                descs[b].wait()

                @pl.when(j + b + NB < nwt)
                def _():
                    g_start(b, j + b + NB)
            return carry

        lax.fori_loop(0, nwt // NB, body, 0)
        for jr in range(nwt - nwt % NB, nwt):
            b = jr % NB
            g_wait(b)
            pltpu.sync_copy(rows[b], acc.at[dbs[b]], add=True)
        plsc.subcore_barrier()

        @pl.when(sid < NS - 1)
        def _():
            pltpu.sync_copy(acc.at[pl.ds(sid * wb, wb)],
                            out_hbm.at[cid, pl.ds(sid * wb, wb)])

        @pl.when(sid == NS - 1)
        def _():
            pltpu.sync_copy(acc.at[pl.ds((NS - 1) * wb, wb_last)],
                            out_hbm.at[cid, pl.ds((NS - 1) * wb, wb_last)])

    return k(packed3, v)


def _tc_input(x, W_in, b_in, W_g1, dp, bn):
    n, d_in = x.shape
    h1 = W_g1.shape[1]
    grid = (n // bn,)

    def body(x_r, wi_r, bi_r, wg_r, d_r, t1s_r, dis_r):
        h0 = jnp.tanh(_dot(x_r[...], wi_r[...]) + bi_r[...])
        d = d_r[0] + d_r[1]
        dis = jnp.where(d > 0, 1.0 / jnp.sqrt(jnp.maximum(d, 1.0)), 0.0)
        t1s_r[...] = _dot(h0, wg_r[...]) * dis
        dis_r[...] = dis

    return pl.pallas_call(
        body,
        grid=grid,
        in_specs=[
            pl.BlockSpec((bn, d_in), lambda i: (i, 0)),
            pl.BlockSpec((d_in, h1), lambda i: (0, 0)),
            pl.BlockSpec((1, h1), lambda i: (0, 0)),
            pl.BlockSpec((h1, h1), lambda i: (0, 0)),
            pl.BlockSpec((NC, bn, 1), lambda i: (0, i, 0)),
        ],
        out_specs=[
            pl.BlockSpec((bn, h1), lambda i: (i, 0)),
            pl.BlockSpec((bn, 1), lambda i: (i, 0)),
        ],
        out_shape=[
            jax.ShapeDtypeStruct((n, h1), F32),
            jax.ShapeDtypeStruct((n, 1), F32),
        ],
    )(x, W_in, b_in, W_g1, dp)


def _tc_middle(p, dis, b_g1, W_l, b_l, W_g2, bn):
    _, n, h1 = p.shape
    h2 = W_g2.shape[1]
    grid = (n // bn,)

    def body(p_r, dis_r, bg1_r, wl_r, bl_r, wg2_r, out_r):
        dis = dis_r[...]
        h1v = jnp.tanh((p_r[0] + p_r[1]) * dis + bg1_r[...])
        h2v = jnp.tanh(_dot(h1v, wl_r[...]) + bl_r[...])
        t2 = _dot(h2v, wg2_r[...]) * dis
        out_r[...] = jnp.concatenate(
            [t2, jnp.zeros((t2.shape[0], h1 - h2), F32)], axis=1)

    return pl.pallas_call(
        body,
        grid=grid,
        in_specs=[
            pl.BlockSpec((NC, bn, h1), lambda i: (0, i, 0)),
            pl.BlockSpec((bn, 1), lambda i: (i, 0)),
            pl.BlockSpec((1, h1), lambda i: (0, 0)),
            pl.BlockSpec((h1, h1), lambda i: (0, 0)),
            pl.BlockSpec((1, h1), lambda i: (0, 0)),
            pl.BlockSpec((h1, h2), lambda i: (0, 0)),
        ],
        out_specs=pl.BlockSpec((bn, h1), lambda i: (i, 0)),
        out_shape=jax.ShapeDtypeStruct((n, h1), F32),
    )(p, dis, b_g1, W_l, b_l, W_g2)


def _tc_embed(q, dis, b_g2, h2, bn):
    _, n, hp = q.shape
    grid = (n // bn,)

    def body(q_r, dis_r, bg2_r, out_r):
        qs = q_r[0, :, :h2] + q_r[1, :, :h2]
        out_r[...] = jnp.tanh(qs * dis_r[...] + bg2_r[...])

    return pl.pallas_call(
        body,
        grid=grid,
        in_specs=[
            pl.BlockSpec((NC, bn, hp), lambda i: (0, i, 0)),
            pl.BlockSpec((bn, 1), lambda i: (i, 0)),
            pl.BlockSpec((1, h2), lambda i: (0, 0)),
        ],
        out_specs=pl.BlockSpec((bn, h2), lambda i: (i, 0)),
        out_shape=jax.ShapeDtypeStruct((n, h2), F32),
    )(q, dis, b_g2)


def _tc_heads(flat, W_p1T, b_p1, W_p2, b_p2, W_v1T, b_v1, W_v2, b_v2, kb):
    ktot = flat.shape[1]
    p_hid = W_p1T.shape[0]
    n_out = W_p2.shape[1]
    nsteps = ktot // kb
    grid = (nsteps,)

    def body(f_r, wp1_r, bp1_r, wp2_r, bp2_r, wv1_r, bv1_r, wv2_r, bv2_r,
             lp_r, lv_r, accp, accv):
        i = pl.program_id(0)

        @pl.when(i == 0)
        def _():
            accp[...] = jnp.zeros_like(accp)
            accv[...] = jnp.zeros_like(accv)

        f = f_r[...]
        accp[...] += _dot_t(f, wp1_r[...])
        accv[...] += _dot_t(f, wv1_r[...])

        @pl.when(i == nsteps - 1)
        def _():
            pi = jnp.tanh(accp[...] + bp1_r[...])
            lp_r[...] = _dot(pi, wp2_r[...]) + bp2_r[...]
            vf = jnp.tanh(accv[...] + bv1_r[...])
            lv_r[...] = _dot(vf, wv2_r[...]) + bv2_r[...]

    return pl.pallas_call(
        body,
        grid=grid,
        in_specs=[
            pl.BlockSpec((1, kb), lambda i: (0, i)),
            pl.BlockSpec((p_hid, kb), lambda i: (0, i)),
            pl.BlockSpec((1, p_hid), lambda i: (0, 0)),
            pl.BlockSpec((p_hid, n_out), lambda i: (0, 0)),
            pl.BlockSpec((1, n_out), lambda i: (0, 0)),
            pl.BlockSpec((p_hid, kb), lambda i: (0, i)),
            pl.BlockSpec((1, p_hid), lambda i: (0, 0)),
            pl.BlockSpec((p_hid, 1), lambda i: (0, 0)),
            pl.BlockSpec((1, 1), lambda i: (0, 0)),
        ],
        out_specs=[
            pl.BlockSpec((1, n_out), lambda i: (0, 0)),
            pl.BlockSpec((1, 1), lambda i: (0, 0)),
        ],
        out_shape=[
            jax.ShapeDtypeStruct((1, n_out), F32),
            jax.ShapeDtypeStruct((1, 1), F32),
        ],
        scratch_shapes=[
            pltpu.VMEM((1, p_hid), F32),
            pltpu.VMEM((1, p_hid), F32),
        ],
    )(flat, W_p1T, b_p1, W_p2, b_p2, W_v1T, b_v1, W_v2, b_v2)


def kernel(x, edge_index, W_in, b_in, W_g1, b_g1, W_l, b_l, W_g2, b_g2,
           W_p1, b_p1, W_p2, b_p2, W_v1, b_v1, W_v2, b_v2):
    n, d_in = x.shape
    e = edge_index.shape[1]
    h1 = W_g1.shape[1]
    h2 = W_g2.shape[1]
    bn = 1000

    nwt = e // (NC * NS * EW)
    packed3 = ((edge_index[1] << 16) | edge_index[0]).reshape(NC * NS, nwt, EW)
    zeros_n = jnp.zeros((n,), F32)

    degp = _sc_degree(packed3, zeros_n, n)
    dp = degp.reshape(NC, n, 1)
    t1s, dis = _tc_input(x, W_in, b_in.reshape(1, -1), W_g1, dp, bn)
    p = _sc_aggregate(packed3, t1s, n, h1)
    t2s = _tc_middle(p, dis, b_g1.reshape(1, -1), W_l,
                     b_l.reshape(1, -1), W_g2, bn)
    q = _sc_aggregate(packed3, t2s, n, h1)
    h3 = _tc_embed(q, dis, b_g2.reshape(1, -1), h2, bn)
    flat = h3.reshape(1, n * h2)
    latent_pi, latent_vf = _tc_heads(flat, W_p1.T, b_p1.reshape(1, -1), W_p2,
                                     b_p2.reshape(1, -1), W_v1.T,
                                     b_v1.reshape(1, -1), W_v2,
                                     b_v2.reshape(1, -1), 16000)
    return latent_pi, latent_vf


# speedup vs baseline: 1.1322x; 1.1322x over previous
import functools

import jax
import jax.numpy as jnp
from jax import lax
from jax.experimental import pallas as pl
from jax.experimental.pallas import tpu as pltpu
from jax.experimental.pallas import tpu_sc as plsc

F32 = jnp.float32
NC, NS = 2, 16
EW = 80

def _dot(a, b):
    return jax.lax.dot_general(a, b, (((1,), (0,)), ((), ())),
                               preferred_element_type=F32)


def _dot_t(a, bT):
    return jax.lax.dot_general(a, bT, (((1,), (1,)), ((), ())),
                               preferred_element_type=F32)


def _sc_degree(packed3, zeros_n, n):
    nwt = packed3.shape[1]

    mesh = plsc.VectorSubcoreMesh(core_axis_name="c", subcore_axis_name="s",
                                  num_cores=NC, num_subcores=NS)

    @functools.partial(
        pl.kernel, mesh=mesh,
        out_type=jax.ShapeDtypeStruct((NC, 1, n), F32),
        scratch_types=[
            pltpu.VMEM((nwt, EW), jnp.int32),
            pltpu.VMEM((EW,), F32),
            pltpu.VMEM((EW,), jnp.int32),
            pltpu.VMEM((EW,), jnp.int32),
            pltpu.VMEM_SHARED((n,), F32),
            pltpu.SemaphoreType.DMA,
            pltpu.SemaphoreType.DMA,
        ],
    )
    def k(pk_hbm, zeros_hbm, out_hbm, pk, ones_v, db0, db1, acc, sem0, sem1):
        cid = lax.axis_index("c")
        sid = lax.axis_index("s")

        for kk in range(EW // 16):
            ones_v[pl.ds(kk * 16, 16)] = jnp.ones((16,), F32)
        wid = cid * NS + sid
        pltpu.sync_copy(pk_hbm.at[wid], pk)

        @pl.when(sid == 0)
        def _():
            pltpu.sync_copy(zeros_hbm, acc)
        plsc.subcore_barrier()

        dbs = (db0, db1)
        sems = (sem0, sem1)

        def s_launch(b, j):
            for kk in range(EW // 16):
                dbs[b][pl.ds(kk * 16, 16)] = lax.shift_right_logical(
                    pk[j, pl.ds(kk * 16, 16)], 16)
            pltpu.async_copy(ones_v, acc.at[dbs[b]], sems[b], add=True)

        def s_wait(b):
            pltpu.make_async_copy(ones_v, acc.at[dbs[b]], sems[b]).wait()

        s_launch(0, 0)
        s_launch(1, 1)

        def body(t, carry):
            j = 2 * t
            for b in range(2):
                s_wait(b)

                @pl.when(j + b + 2 < nwt)
                def _():
                    s_launch(b, j + b + 2)
            return carry

        lax.fori_loop(0, nwt // 2, body, 0)
        if nwt % 2 == 1:
            s_wait(0)
        plsc.subcore_barrier()

        @pl.when(sid == 0)
        def _():
            pltpu.sync_copy(acc, out_hbm.at[cid, 0])

    return k(packed3, zeros_n)


def _sc_aggregate(packed3, v, n, h):
    nwt = packed3.shape[1]
    wb = (n // NS) & ~7
    wb_last = n - (NS - 1) * wb
    NB = 3

    mesh = plsc.VectorSubcoreMesh(core_axis_name="c", subcore_axis_name="s",
                                  num_cores=NC, num_subcores=NS)

    @functools.partial(
        pl.kernel, mesh=mesh,
        out_type=jax.ShapeDtypeStruct((NC, n, h), F32),
        scratch_types=[
            pltpu.VMEM((nwt, EW), jnp.int32),
            *[pltpu.VMEM((EW,), jnp.int32) for _ in range(2 * NB)],
            *[pltpu.VMEM((EW, h), F32) for _ in range(NB)],
            pltpu.VMEM_SHARED((n, h), F32),
            *[pltpu.SemaphoreType.DMA for _ in range(2 * NB)],
        ],
    )
    def k(pk_hbm, v_hbm, out_hbm, pk, *bufs):
        sbs = bufs[0:2 * NB:2]
        dbs = bufs[1:2 * NB:2]
        rows = bufs[2 * NB:3 * NB]
        acc = bufs[3 * NB]
        sems = bufs[3 * NB + 1:3 * NB + 1 + NB]
        ssems = bufs[3 * NB + 1 + NB:]
        cid = lax.axis_index("c")
        sid = lax.axis_index("s")
        wid = cid * NS + sid
        pltpu.sync_copy(pk_hbm.at[wid], pk)

        def zbody(r, c):
            for kk in range(h // 16):
                rows[0][r, pl.ds(kk * 16, 16)] = jnp.zeros((16,), F32)
            return c

        lax.fori_loop(0, EW, zbody, 0)
        zc = 48
        base = sid * wb
        for c in range(wb // zc):
            pltpu.sync_copy(rows[0].at[pl.ds(0, zc)],
                            acc.at[pl.ds(base + c * zc, zc)])
        if wb % zc:
            pltpu.sync_copy(rows[0].at[pl.ds(0, wb % zc)],
                            acc.at[pl.ds(base + (wb // zc) * zc, wb % zc)])

        @pl.when(sid == NS - 1)
        def _():
            ext = wb_last - wb
            pltpu.sync_copy(rows[0].at[pl.ds(0, ext)],
                            acc.at[pl.ds(NS * wb, ext)])
        plsc.subcore_barrier()

        def unpack(j, sb, db):
            for kk in range(EW // 16):
                pv = pk[j, pl.ds(kk * 16, 16)]
                sb[pl.ds(kk * 16, 16)] = pv & 0xFFFF
                db[pl.ds(kk * 16, 16)] = lax.shift_right_logical(pv, 16)

        def g_start(b, j):
            unpack(j, sbs[b], dbs[b])
            pltpu.make_async_copy(v_hbm.at[sbs[b]], rows[b], sems[b]).start()

        def g_wait(b):
            pltpu.make_async_copy(v_hbm.at[sbs[b]], rows[b], sems[b]).wait()

        for b in range(NB):
            g_start(b, b)

        def body(t, carry):
            j = NB * t
            for b in range(NB):
                g_wait(b)
                pltpu.sync_copy(rows[b], acc.at[dbs[b]], add=True)

                @pl.when(j + b + NB < nwt)
                def _():
                    g_start(b, j + b + NB)
            return carry

        lax.fori_loop(0, nwt // NB, body, 0)
        for jr in range(nwt - nwt % NB, nwt):
            b = jr % NB
            g_wait(b)
            pltpu.sync_copy(rows[b], acc.at[dbs[b]], add=True)
        plsc.subcore_barrier()

        @pl.when(sid < NS - 1)
        def _():
            pltpu.sync_copy(acc.at[pl.ds(sid * wb, wb)],
                            out_hbm.at[cid, pl.ds(sid * wb, wb)])

        @pl.when(sid == NS - 1)
        def _():
            pltpu.sync_copy(acc.at[pl.ds((NS - 1) * wb, wb_last)],
                            out_hbm.at[cid, pl.ds((NS - 1) * wb, wb_last)])

    return k(packed3, v)


def _tc_input(x, W_in, b_in, W_g1, dp, bn):
    n, d_in = x.shape
    h1 = W_g1.shape[1]
    grid = (n // bn,)

    def body(x_r, wi_r, bi_r, wg_r, d_r, t1s_r, dis_r):
        h0 = jnp.tanh(_dot(x_r[...], wi_r[...]) + bi_r[...])
        d = d_r[0] + d_r[1]
        dis = jnp.where(d > 0, 1.0 / jnp.sqrt(jnp.maximum(d, 1.0)), 0.0)
        t1s_r[...] = _dot(h0, wg_r[...]) * dis
        dis_r[...] = dis

    return pl.pallas_call(
        body,
        grid=grid,
        in_specs=[
            pl.BlockSpec((bn, d_in), lambda i: (i, 0)),
            pl.BlockSpec((d_in, h1), lambda i: (0, 0)),
            pl.BlockSpec((1, h1), lambda i: (0, 0)),
            pl.BlockSpec((h1, h1), lambda i: (0, 0)),
            pl.BlockSpec((NC, bn, 1), lambda i: (0, i, 0)),
        ],
        out_specs=[
            pl.BlockSpec((bn, h1), lambda i: (i, 0)),
            pl.BlockSpec((bn, 1), lambda i: (i, 0)),
        ],
        out_shape=[
            jax.ShapeDtypeStruct((n, h1), F32),
            jax.ShapeDtypeStruct((n, 1), F32),
        ],
    )(x, W_in, b_in, W_g1, dp)


def _tc_middle(p, dis, b_g1, W_l, b_l, W_g2, bn):
    _, n, h1 = p.shape
    h2 = W_g2.shape[1]
    grid = (n // bn,)

    def body(p_r, dis_r, bg1_r, wl_r, bl_r, wg2_r, out_r):
        dis = dis_r[...]
        h1v = jnp.tanh((p_r[0] + p_r[1]) * dis + bg1_r[...])
        h2v = jnp.tanh(_dot(h1v, wl_r[...]) + bl_r[...])
        t2 = _dot(h2v, wg2_r[...]) * dis
        out_r[...] = jnp.concatenate(
            [t2, jnp.zeros((t2.shape[0], h1 - h2), F32)], axis=1)

    return pl.pallas_call(
        body,
        grid=grid,
        in_specs=[
            pl.BlockSpec((NC, bn, h1), lambda i: (0, i, 0)),
            pl.BlockSpec((bn, 1), lambda i: (i, 0)),
            pl.BlockSpec((1, h1), lambda i: (0, 0)),
            pl.BlockSpec((h1, h1), lambda i: (0, 0)),
            pl.BlockSpec((1, h1), lambda i: (0, 0)),
            pl.BlockSpec((h1, h2), lambda i: (0, 0)),
        ],
        out_specs=pl.BlockSpec((bn, h1), lambda i: (i, 0)),
        out_shape=jax.ShapeDtypeStruct((n, h1), F32),
    )(p, dis, b_g1, W_l, b_l, W_g2)


def _tc_embed(q, dis, b_g2, h2, bn):
    _, n, hp = q.shape
    grid = (n // bn,)

    def body(q_r, dis_r, bg2_r, out_r):
        qs = q_r[0, :, :h2] + q_r[1, :, :h2]
        out_r[...] = jnp.tanh(qs * dis_r[...] + bg2_r[...])

    return pl.pallas_call(
        body,
        grid=grid,
        in_specs=[
            pl.BlockSpec((NC, bn, hp), lambda i: (0, i, 0)),
            pl.BlockSpec((bn, 1), lambda i: (i, 0)),
            pl.BlockSpec((1, h2), lambda i: (0, 0)),
        ],
        out_specs=pl.BlockSpec((bn, h2), lambda i: (i, 0)),
        out_shape=jax.ShapeDtypeStruct((n, h2), F32),
    )(q, dis, b_g2)


def _tc_heads(flat, W_p1T, b_p1, W_p2, b_p2, W_v1T, b_v1, W_v2, b_v2, kb):
    ktot = flat.shape[1]
    p_hid = W_p1T.shape[0]
    n_out = W_p2.shape[1]
    nsteps = ktot // kb
    grid = (nsteps,)

    def body(f_r, wp1_r, bp1_r, wp2_r, bp2_r, wv1_r, bv1_r, wv2_r, bv2_r,
             lp_r, lv_r, accp, accv):
        i = pl.program_id(0)

        @pl.when(i == 0)
        def _():
            accp[...] = jnp.zeros_like(accp)
            accv[...] = jnp.zeros_like(accv)

        f = f_r[...]
        accp[...] += _dot_t(f, wp1_r[...])
        accv[...] += _dot_t(f, wv1_r[...])

        @pl.when(i == nsteps - 1)
        def _():
            pi = jnp.tanh(accp[...] + bp1_r[...])
            lp_r[...] = _dot(pi, wp2_r[...]) + bp2_r[...]
            vf = jnp.tanh(accv[...] + bv1_r[...])
            lv_r[...] = _dot(vf, wv2_r[...]) + bv2_r[...]

    return pl.pallas_call(
        body,
        grid=grid,
        in_specs=[
            pl.BlockSpec((1, kb), lambda i: (0, i)),
            pl.BlockSpec((p_hid, kb), lambda i: (0, i)),
            pl.BlockSpec((1, p_hid), lambda i: (0, 0)),
            pl.BlockSpec((p_hid, n_out), lambda i: (0, 0)),
            pl.BlockSpec((1, n_out), lambda i: (0, 0)),
            pl.BlockSpec((p_hid, kb), lambda i: (0, i)),
            pl.BlockSpec((1, p_hid), lambda i: (0, 0)),
            pl.BlockSpec((p_hid, 1), lambda i: (0, 0)),
            pl.BlockSpec((1, 1), lambda i: (0, 0)),
        ],
        out_specs=[
            pl.BlockSpec((1, n_out), lambda i: (0, 0)),
            pl.BlockSpec((1, 1), lambda i: (0, 0)),
        ],
        out_shape=[
            jax.ShapeDtypeStruct((1, n_out), F32),
            jax.ShapeDtypeStruct((1, 1), F32),
        ],
        scratch_shapes=[
            pltpu.VMEM((1, p_hid), F32),
            pltpu.VMEM((1, p_hid), F32),
        ],
    )(flat, W_p1T, b_p1, W_p2, b_p2, W_v1T, b_v1, W_v2, b_v2)


def kernel(x, edge_index, W_in, b_in, W_g1, b_g1, W_l, b_l, W_g2, b_g2,
           W_p1, b_p1, W_p2, b_p2, W_v1, b_v1, W_v2, b_v2):
    n, d_in = x.shape
    e = edge_index.shape[1]
    h1 = W_g1.shape[1]
    h2 = W_g2.shape[1]
    bn = 1000

    nwt = e // (NC * NS * EW)
    packed3 = ((edge_index[1] << 16) | edge_index[0]).reshape(NC * NS, nwt, EW)
    zeros_n = jnp.zeros((n,), F32)

    degp = _sc_degree(packed3, zeros_n, n)
    dp = degp.reshape(NC, n, 1)
    t1s, dis = _tc_input(x, W_in, b_in.reshape(1, -1), W_g1, dp, bn)
    p = _sc_aggregate(packed3, t1s, n, h1)
    t2s = _tc_middle(p, dis, b_g1.reshape(1, -1), W_l,
                     b_l.reshape(1, -1), W_g2, bn)
    q = _sc_aggregate(packed3, t2s, n, h1)
    h3 = _tc_embed(q, dis, b_g2.reshape(1, -1), h2, bn)
    flat = h3.reshape(1, n * h2)
    latent_pi, latent_vf = _tc_heads(flat, W_p1.T, b_p1.reshape(1, -1), W_p2,
                                     b_p2.reshape(1, -1), W_v1.T,
                                     b_v1.reshape(1, -1), W_v2,
                                     b_v2.reshape(1, -1), 16000)
    return latent_pi, latent_vf
